# baseline (speedup 1.0000x reference)
import functools

import jax
import jax.numpy as jnp
from jax import lax
from jax.experimental import pallas as pl
from jax.experimental.pallas import tpu as pltpu
from jax.experimental.pallas import tpu_sc as plsc

_NC = 2
_NS = 16
_NW = _NC * _NS
_CH = 56


@functools.lru_cache(maxsize=None)
def _make_gather(batch: int, seq: int, dim: int):
    n_idx = batch * seq
    bpw = n_idx // _NW
    spw = seq // bpw
    sizes = [_CH] * (bpw // _CH)
    if bpw % _CH:
        sizes.append(bpw % _CH)
    offs = [sum(sizes[:i]) for i in range(len(sizes))]
    mesh = plsc.VectorSubcoreMesh(core_axis_name="c", subcore_axis_name="s")
    nbuf = 2

    @functools.partial(
        pl.kernel,
        out_type=jax.ShapeDtypeStruct((n_idx, dim), jnp.float32),
        mesh=mesh,
        scratch_types=[
            pltpu.VMEM((bpw,), jnp.int32),
            [pltpu.VMEM((_CH, dim), jnp.float32)] * nbuf,
            [pltpu.SemaphoreType.DMA] * nbuf,
            [pltpu.SemaphoreType.DMA] * nbuf,
        ],
    )
    def grab(table_hbm, idx_hbm, out_hbm, idx_v, bufs, gsems, ssems):
        wid = lax.axis_index("s") * _NC + lax.axis_index("c")
        base = wid * bpw
        pltpu.sync_copy(idx_hbm.at[wid // spw, pl.ds((wid % spw) * bpw, bpw)],
                        idx_v)

        def start_gather(c):
            return pltpu.async_copy(
                table_hbm.at[idx_v.at[pl.ds(offs[c], sizes[c])]],
                bufs[c % nbuf].at[pl.ds(0, sizes[c])], gsems[c % nbuf])

        def start_store(c):
            return pltpu.async_copy(
                bufs[c % nbuf].at[pl.ds(0, sizes[c])],
                out_hbm.at[pl.ds(base + offs[c], sizes[c])],
                ssems[c % nbuf])

        nchunk = len(sizes)
        gathers = [None] * nbuf
        stores = [None] * nbuf
        for c in range(min(nbuf - 1, nchunk)):
            gathers[c % nbuf] = start_gather(c)
        for c in range(nchunk):
            b = c % nbuf
            gathers[b].wait()
            stores[b] = start_store(c)
            n = c + nbuf - 1
            if n < nchunk:
                nb = n % nbuf
                if stores[nb] is not None:
                    stores[nb].wait()
                    stores[nb] = None
                gathers[nb] = start_gather(n)
        for s in stores:
            if s is not None:
                s.wait()

    return grab


def kernel(position, position_encoding):
    batch, seq = position.shape
    dim = position_encoding.shape[1]
    idx = position.astype(jnp.int32)
    table = position_encoding.astype(jnp.float32)
    out = _make_gather(batch, seq, dim)(table, idx)
    return out.reshape(batch, seq, dim)


# speedup vs baseline: 1.0383x; 1.0383x over previous
import functools

import jax
import jax.numpy as jnp
from jax import lax
from jax.experimental import pallas as pl
from jax.experimental.pallas import tpu as pltpu
from jax.experimental.pallas import tpu_sc as plsc

_NC = 2
_NS = 16
_NW = _NC * _NS
_CH = 16


@functools.lru_cache(maxsize=None)
def _make_gather(batch: int, seq: int, dim: int):
    n_idx = batch * seq
    bpw = n_idx // _NW
    spw = seq // bpw
    sizes = [_CH] * (bpw // _CH)
    if bpw % _CH:
        sizes.append(bpw % _CH)
    offs = [sum(sizes[:i]) for i in range(len(sizes))]
    mesh = plsc.VectorSubcoreMesh(core_axis_name="c", subcore_axis_name="s")
    nbuf = 4

    @functools.partial(
        pl.kernel,
        out_type=jax.ShapeDtypeStruct((n_idx, dim), jnp.float32),
        mesh=mesh,
        scratch_types=[
            pltpu.VMEM((bpw,), jnp.int32),
            [pltpu.VMEM((_CH, dim), jnp.float32)] * nbuf,
            [pltpu.SemaphoreType.DMA] * nbuf,
            [pltpu.SemaphoreType.DMA] * nbuf,
        ],
    )
    def grab(table_hbm, idx_hbm, out_hbm, idx_v, bufs, gsems, ssems):
        wid = lax.axis_index("s") * _NC + lax.axis_index("c")
        base = wid * bpw
        pltpu.sync_copy(idx_hbm.at[wid // spw, pl.ds((wid % spw) * bpw, bpw)],
                        idx_v)

        def start_gather(c):
            return pltpu.async_copy(
                table_hbm.at[idx_v.at[pl.ds(offs[c], sizes[c])]],
                bufs[c % nbuf].at[pl.ds(0, sizes[c])], gsems[c % nbuf])

        def start_store(c):
            return pltpu.async_copy(
                bufs[c % nbuf].at[pl.ds(0, sizes[c])],
                out_hbm.at[pl.ds(base + offs[c], sizes[c])],
                ssems[c % nbuf])

        nchunk = len(sizes)
        gathers = [None] * nbuf
        stores = [None] * nbuf
        for c in range(min(nbuf - 1, nchunk)):
            gathers[c % nbuf] = start_gather(c)
        for c in range(nchunk):
            b = c % nbuf
            gathers[b].wait()
            stores[b] = start_store(c)
            n = c + nbuf - 1
            if n < nchunk:
                nb = n % nbuf
                if stores[nb] is not None:
                    stores[nb].wait()
                    stores[nb] = None
                gathers[nb] = start_gather(n)
        for s in stores:
            if s is not None:
                s.wait()

    return grab


def kernel(position, position_encoding):
    batch, seq = position.shape
    dim = position_encoding.shape[1]
    idx = position.astype(jnp.int32)
    table = position_encoding.astype(jnp.float32)
    out = _make_gather(batch, seq, dim)(table, idx)
    return out.reshape(batch, seq, dim)


# speedup vs baseline: 1.0622x; 1.0230x over previous
import functools

import jax
import jax.numpy as jnp
from jax import lax
from jax.experimental import pallas as pl
from jax.experimental.pallas import tpu as pltpu
from jax.experimental.pallas import tpu_sc as plsc

_NC = 2
_NS = 16
_NW = _NC * _NS
_CH = 16


@functools.lru_cache(maxsize=None)
def _make_gather(batch: int, seq: int, dim: int):
    n_idx = batch * seq
    bpw = n_idx // _NW
    spw = seq // bpw
    sizes = [_CH] * (bpw // _CH)
    if bpw % _CH:
        sizes.append(bpw % _CH)
    offs = [sum(sizes[:i]) for i in range(len(sizes))]
    mesh = plsc.VectorSubcoreMesh(core_axis_name="c", subcore_axis_name="s")
    nbuf = 6

    @functools.partial(
        pl.kernel,
        out_type=jax.ShapeDtypeStruct((n_idx, dim), jnp.float32),
        mesh=mesh,
        scratch_types=[
            pltpu.VMEM((bpw,), jnp.int32),
            [pltpu.VMEM((_CH, dim), jnp.float32)] * nbuf,
            [pltpu.SemaphoreType.DMA] * nbuf,
            [pltpu.SemaphoreType.DMA] * nbuf,
        ],
    )
    def grab(table_hbm, idx_hbm, out_hbm, idx_v, bufs, gsems, ssems):
        wid = lax.axis_index("s") * _NC + lax.axis_index("c")
        base = wid * bpw
        pltpu.sync_copy(idx_hbm.at[wid // spw, pl.ds((wid % spw) * bpw, bpw)],
                        idx_v)

        def start_gather(c):
            return pltpu.async_copy(
                table_hbm.at[idx_v.at[pl.ds(offs[c], sizes[c])]],
                bufs[c % nbuf].at[pl.ds(0, sizes[c])], gsems[c % nbuf])

        def start_store(c):
            return pltpu.async_copy(
                bufs[c % nbuf].at[pl.ds(0, sizes[c])],
                out_hbm.at[pl.ds(base + offs[c], sizes[c])],
                ssems[c % nbuf])

        nchunk = len(sizes)
        gathers = [None] * nbuf
        stores = [None] * nbuf
        for c in range(min(nbuf - 1, nchunk)):
            gathers[c % nbuf] = start_gather(c)
        for c in range(nchunk):
            b = c % nbuf
            gathers[b].wait()
            stores[b] = start_store(c)
            n = c + nbuf - 1
            if n < nchunk:
                nb = n % nbuf
                if stores[nb] is not None:
                    stores[nb].wait()
                    stores[nb] = None
                gathers[nb] = start_gather(n)
        for s in stores:
            if s is not None:
                s.wait()

    return grab


def kernel(position, position_encoding):
    batch, seq = position.shape
    dim = position_encoding.shape[1]
    idx = position.astype(jnp.int32)
    table = position_encoding.astype(jnp.float32)
    out = _make_gather(batch, seq, dim)(table, idx)
    return out.reshape(batch, seq, dim)
